# baseline (speedup 1.0000x reference)
import functools

import numpy as np
import jax
import jax.numpy as jnp
from jax import lax
from jax.experimental import pallas as pl
from jax.experimental.pallas import tpu as pltpu
from jax.experimental.pallas import tpu_sc as plsc

_B, _T, _C = 64, 3584, 32
_NSEG = 7
_SEG = _T // _NSEG
_ROWS = _B * _C
_NWORK = 32
_RPW = _ROWS // _NWORK
_RB = 4
_NBLK = _RPW // _RB
_LANES = 16
_GPR = _T // 32
_WPR = _T // 2


def _threefry2x32(k1, k2, x0, x1):
    rot_a = (13, 15, 26, 6)
    rot_b = (17, 29, 16, 24)
    ks = [np.uint32(k1), np.uint32(k2), np.uint32(k1 ^ k2 ^ np.uint32(0x1BD11BDA))]
    x0 = x0 + ks[0]
    x1 = x1 + ks[1]
    rots = (rot_a, rot_b, rot_a, rot_b, rot_a)
    for i in range(5):
        for r in rots[i]:
            x0 = x0 + x1
            x1 = (x1 << np.uint32(r)) | (x1 >> np.uint32(32 - r))
            x1 = x0 ^ x1
        x0 = x0 + ks[(i + 1) % 3]
        x1 = x1 + ks[(i + 2) % 3] + np.uint32(i + 1)
    return x0, x1


def _build_packed_indices() -> np.ndarray:
    size = _B * _NSEG * _SEG * _C
    i = np.arange(size, dtype=np.uint64)
    hi = (i >> np.uint64(32)).astype(np.uint32)
    lo = (i & np.uint64(0xFFFFFFFF)).astype(np.uint32)
    with np.errstate(over="ignore"):
        o0, o1 = _threefry2x32(np.uint32(0), np.uint32(42), hi, lo)
    bits = o0 ^ o1
    fb = (bits >> np.uint32(9)) | np.uint32(0x3F800000)
    u = (fb.view(np.float32) - np.float32(1.0)).reshape(_B, _NSEG, _SEG, _C)
    perm = np.argsort(u, axis=2, kind="stable").astype(np.int32)
    col = perm + (np.arange(_NSEG, dtype=np.int32) * _SEG)[None, :, None, None]
    rowpos = np.ascontiguousarray(col.transpose(0, 3, 1, 2)).reshape(_ROWS, _GPR, 32)
    packed = rowpos[..., 0:_LANES] | (rowpos[..., _LANES:32] << 16)
    return np.ascontiguousarray(packed.reshape(_ROWS, _WPR))


_IDX = _build_packed_indices()


def _permute_body(a_hbm, idx_hbm, out_hbm,
                  a0, a1, i0, i1, o0, o1,
                  sa0, sa1, si0, si1, so0, so1,
                  n_rows_per_worker):
    wid = lax.axis_index("s") * 2 + lax.axis_index("c")
    a_bufs, i_bufs, o_bufs = (a0, a1), (i0, i1), (o0, o1)
    sa, si, so = (sa0, sa1), (si0, si1), (so0, so1)
    _NBLK = n_rows_per_worker // _RB
    row0w = wid * n_rows_per_worker

    def start_in(t):
        p = t & 1
        r0 = row0w + t * _RB
        ha = pltpu.async_copy(a_hbm.at[pl.ds(r0, _RB)], a_bufs[p], sa[p])
        hi_ = pltpu.async_copy(idx_hbm.at[pl.ds(r0, _RB)], i_bufs[p], si[p])
        return ha, hi_

    in_h = {0: start_in(0)}
    out_h = {}
    for t in range(_NBLK):
        p = t & 1
        if t + 1 < _NBLK:
            in_h[t + 1] = start_in(t + 1)
        ha, hi_ = in_h.pop(t)
        ha.wait()
        hi_.wait()
        if t >= 2:
            out_h.pop(t - 2).wait()
        a_v, i_v, o_v = a_bufs[p], i_bufs[p], o_bufs[p]

        for r in range(_RB):
            rvec = jnp.full((_LANES,), r, jnp.int32)

            @plsc.parallel_loop(0, _GPR, unroll=8)
            def _gather(g):
                w = i_v[r, pl.ds(g * _LANES, _LANES)]
                c_lo = w & 0xFFFF
                c_hi = lax.shift_right_logical(w, 16)
                o_v[r, pl.ds(g * 32, _LANES)] = plsc.load_gather(a_v, [rvec, c_lo])
                o_v[r, pl.ds(g * 32 + _LANES, _LANES)] = plsc.load_gather(
                    a_v, [rvec, c_hi])

        r0 = row0w + t * _RB
        out_h[t] = pltpu.async_copy(o_v, out_hbm.at[pl.ds(r0, _RB)], so[p])

    for t in sorted(out_h):
        out_h.pop(t).wait()


_NCHUNK = 2
_ROWS_PC = _ROWS // _NCHUNK


def kernel(A):
    a2 = jnp.transpose(A, (0, 2, 1)).reshape(_ROWS, _T)
    idx = jnp.asarray(_IDX)
    call = pl.kernel(
        functools.partial(_permute_body,
                          n_rows_per_worker=_ROWS_PC // _NWORK),
        out_type=jax.ShapeDtypeStruct((_ROWS_PC, _T), jnp.float32),
        mesh=plsc.VectorSubcoreMesh(core_axis_name="c", subcore_axis_name="s"),
        scratch_types=[
            pltpu.VMEM((_RB, _T), jnp.float32),
            pltpu.VMEM((_RB, _T), jnp.float32),
            pltpu.VMEM((_RB, _WPR), jnp.int32),
            pltpu.VMEM((_RB, _WPR), jnp.int32),
            pltpu.VMEM((_RB, _T), jnp.float32),
            pltpu.VMEM((_RB, _T), jnp.float32),
            pltpu.SemaphoreType.DMA,
            pltpu.SemaphoreType.DMA,
            pltpu.SemaphoreType.DMA,
            pltpu.SemaphoreType.DMA,
            pltpu.SemaphoreType.DMA,
            pltpu.SemaphoreType.DMA,
        ],
        compiler_params=pltpu.CompilerParams(
            needs_layout_passes=False, use_tc_tiling_on_sc=False),
    )
    outs = [call(a2[h * _ROWS_PC:(h + 1) * _ROWS_PC],
                 idx[h * _ROWS_PC:(h + 1) * _ROWS_PC])
            for h in range(_NCHUNK)]
    out2 = jnp.concatenate(outs, axis=0)
    return jnp.transpose(out2.reshape(_B, _C, _T), (0, 2, 1))


# speedup vs baseline: 1.2907x; 1.2907x over previous
import numpy as np
import jax
import jax.numpy as jnp
from jax import lax
from jax.experimental import pallas as pl
from jax.experimental.pallas import tpu as pltpu
from jax.experimental.pallas import tpu_sc as plsc

_B, _T, _C = 64, 3584, 32
_NSEG = 7
_SEG = _T // _NSEG
_ROWS = _B * _C
_NWORK = 32
_RPW = _ROWS // _NWORK
_RB = 4
_NBLK = _RPW // _RB
_LANES = 16
_GPR = _T // 32
_WPR = _T // 2


def _threefry2x32(k1, k2, x0, x1):
    rot_a = (13, 15, 26, 6)
    rot_b = (17, 29, 16, 24)
    ks = [np.uint32(k1), np.uint32(k2), np.uint32(k1 ^ k2 ^ np.uint32(0x1BD11BDA))]
    x0 = x0 + ks[0]
    x1 = x1 + ks[1]
    rots = (rot_a, rot_b, rot_a, rot_b, rot_a)
    for i in range(5):
        for r in rots[i]:
            x0 = x0 + x1
            x1 = (x1 << np.uint32(r)) | (x1 >> np.uint32(32 - r))
            x1 = x0 ^ x1
        x0 = x0 + ks[(i + 1) % 3]
        x1 = x1 + ks[(i + 2) % 3] + np.uint32(i + 1)
    return x0, x1


def _build_packed_indices() -> np.ndarray:
    size = _B * _NSEG * _SEG * _C
    i = np.arange(size, dtype=np.uint64)
    hi = (i >> np.uint64(32)).astype(np.uint32)
    lo = (i & np.uint64(0xFFFFFFFF)).astype(np.uint32)
    with np.errstate(over="ignore"):
        o0, o1 = _threefry2x32(np.uint32(0), np.uint32(42), hi, lo)
    bits = o0 ^ o1
    fb = (bits >> np.uint32(9)) | np.uint32(0x3F800000)
    u = (fb.view(np.float32) - np.float32(1.0)).reshape(_B, _NSEG, _SEG, _C)
    perm = np.argsort(u, axis=2, kind="stable").astype(np.int32)
    col = perm + (np.arange(_NSEG, dtype=np.int32) * _SEG)[None, :, None, None]
    rowpos = np.ascontiguousarray(col.transpose(0, 3, 1, 2)).reshape(_ROWS, _GPR, 32)
    packed = rowpos[..., 0:_LANES] | (rowpos[..., _LANES:32] << 16)
    return np.ascontiguousarray(packed.reshape(_ROWS, _WPR))


_IDX = _build_packed_indices()


def _permute_body(a_hbm, idx_hbm, out_hbm,
                  a0, a1, i0, i1, o0, o1,
                  sa0, sa1, si0, si1, so0, so1):
    wid = lax.axis_index("s") * 2 + lax.axis_index("c")
    a_bufs, i_bufs, o_bufs = (a0, a1), (i0, i1), (o0, o1)
    sa, si, so = (sa0, sa1), (si0, si1), (so0, so1)
    row0w = wid * _RPW

    def start_in(t):
        p = t & 1
        r0 = row0w + t * _RB
        ha = pltpu.async_copy(a_hbm.at[pl.ds(r0, _RB)], a_bufs[p], sa[p])
        hi_ = pltpu.async_copy(idx_hbm.at[pl.ds(r0, _RB)], i_bufs[p], si[p])
        return ha, hi_

    in_h = {0: start_in(0)}
    out_h = {}
    for t in range(_NBLK):
        p = t & 1
        if t + 1 < _NBLK:
            in_h[t + 1] = start_in(t + 1)
        ha, hi_ = in_h.pop(t)
        ha.wait()
        hi_.wait()
        if t >= 2:
            out_h.pop(t - 2).wait()
        a_v, i_v, o_v = a_bufs[p], i_bufs[p], o_bufs[p]

        for r in range(_RB):
            rvec = jnp.full((_LANES,), r, jnp.int32)

            @plsc.parallel_loop(0, _GPR, unroll=8)
            def _gather(g):
                w = i_v[r, pl.ds(g * _LANES, _LANES)]
                c_lo = w & 0xFFFF
                c_hi = lax.shift_right_logical(w, 16)
                o_v[r, pl.ds(g * 32, _LANES)] = plsc.load_gather(a_v, [rvec, c_lo])
                o_v[r, pl.ds(g * 32 + _LANES, _LANES)] = plsc.load_gather(
                    a_v, [rvec, c_hi])

        r0 = row0w + t * _RB
        out_h[t] = pltpu.async_copy(o_v, out_hbm.at[pl.ds(r0, _RB)], so[p])

    for t in sorted(out_h):
        out_h.pop(t).wait()


def kernel(A):
    a2 = jnp.transpose(A, (0, 2, 1)).reshape(_ROWS, _T)
    idx = jnp.asarray(_IDX)
    call = pl.kernel(
        _permute_body,
        out_type=jax.ShapeDtypeStruct((_ROWS, _T), jnp.float32),
        mesh=plsc.VectorSubcoreMesh(core_axis_name="c", subcore_axis_name="s"),
        scratch_types=[
            pltpu.VMEM((_RB, _T), jnp.float32),
            pltpu.VMEM((_RB, _T), jnp.float32),
            pltpu.VMEM((_RB, _WPR), jnp.int32),
            pltpu.VMEM((_RB, _WPR), jnp.int32),
            pltpu.VMEM((_RB, _T), jnp.float32),
            pltpu.VMEM((_RB, _T), jnp.float32),
            pltpu.SemaphoreType.DMA,
            pltpu.SemaphoreType.DMA,
            pltpu.SemaphoreType.DMA,
            pltpu.SemaphoreType.DMA,
            pltpu.SemaphoreType.DMA,
            pltpu.SemaphoreType.DMA,
        ],
        compiler_params=pltpu.CompilerParams(
            needs_layout_passes=False, use_tc_tiling_on_sc=False),
    )
    out2 = call(a2, idx)
    return jnp.transpose(out2.reshape(_B, _C, _T), (0, 2, 1))
